# Initial kernel scaffold; baseline (speedup 1.0000x reference)
import jax
import jax.numpy as jnp
from jax.experimental import pallas as pl


def kernel(states, action, eW1, eb1, eW2, eb2, eg, ebt, eW3, eb3, nW1, nb1, nW2, nb2, ng, nbt, nW3, nb3):
    raise NotImplementedError("write your pallas kernel here")


# speedup vs baseline: 20.8343x; 20.8343x over previous
import jax
import jax.numpy as jnp
from jax.experimental import pallas as pl
from jax.experimental.pallas import tpu as pltpu

_O, _OBS, _ACT, _H = 32, 32, 4, 64
_BB = 8


def _fused(x_ref, act_ref,
           eW1a_ref, eW1b_ref, eb1_ref, eW2_ref, eb2_ref, eg_ref, ebt_ref,
           eW3_ref, eb3_ref,
           nW1x_ref, nW4_ref, nW1g_ref, nb1_ref, nW2_ref, nb2_ref, ng_ref,
           nbt_ref, nW3_ref, nb3_ref, out_ref):
    bb = x_ref.shape[0]
    O, OBS, ACT, H = _O, _OBS, _ACT, _H
    f32 = jnp.float32

    x = x_ref[...].reshape(bb * O, OBS)

    a_src = jnp.dot(x, eW1a_ref[...], preferred_element_type=f32)
    b_dst = jnp.dot(x, eW1b_ref[...], preferred_element_type=f32)
    h1 = (a_src.reshape(bb, O, 1, H) + b_dst.reshape(bb, 1, O, H)
          + eb1_ref[...].reshape(1, 1, 1, H))
    h1 = jnp.maximum(h1, 0.0).reshape(bb * O * O, H)

    h2 = jnp.dot(h1, eW2_ref[...], preferred_element_type=f32) + eb2_ref[...]
    mu = jnp.mean(h2, axis=-1, keepdims=True)
    var = jnp.mean((h2 - mu) ** 2, axis=-1, keepdims=True)
    h2 = (h2 - mu) * jax.lax.rsqrt(var + 1e-5) * eg_ref[...] + ebt_ref[...]
    h2 = jnp.maximum(h2, 0.0)

    e = jnp.dot(h2, eW3_ref[...], preferred_element_type=f32) + eb3_ref[...]
    e4 = e.reshape(bb, O, O, H)

    ii = jax.lax.broadcasted_iota(jnp.int32, (1, O, O, 1), 1)
    jj = jax.lax.broadcasted_iota(jnp.int32, (1, O, O, 1), 2)
    mask = (ii != jj).astype(f32)
    agg = jnp.sum(e4 * mask, axis=2).reshape(bb * O, H)

    act = act_ref[...]
    obj_sel = (act // ACT ==
               jax.lax.broadcasted_iota(jnp.int32, (bb, O), 1)).astype(f32)
    mod = act[:, :1] % ACT
    wrow = jnp.zeros((bb, H), f32)
    for k in range(ACT):
        wrow = wrow + (mod == k).astype(f32) * nW4_ref[k:k + 1, :]
    contrib = (obj_sel.reshape(bb, O, 1) * wrow.reshape(bb, 1, H))
    contrib = contrib.reshape(bb * O, H)

    n1 = (jnp.dot(x, nW1x_ref[...], preferred_element_type=f32)
          + jnp.dot(agg, nW1g_ref[...], preferred_element_type=f32)
          + contrib + nb1_ref[...])
    n1 = jnp.maximum(n1, 0.0)
    n2 = jnp.dot(n1, nW2_ref[...], preferred_element_type=f32) + nb2_ref[...]
    mu2 = jnp.mean(n2, axis=-1, keepdims=True)
    var2 = jnp.mean((n2 - mu2) ** 2, axis=-1, keepdims=True)
    n2 = (n2 - mu2) * jax.lax.rsqrt(var2 + 1e-5) * ng_ref[...] + nbt_ref[...]
    n2 = jnp.maximum(n2, 0.0)
    out = jnp.dot(n2, nW3_ref[...], preferred_element_type=f32) + nb3_ref[...]
    out_ref[...] = out.reshape(bb, O, OBS)


def kernel(states, action, eW1, eb1, eW2, eb2, eg, ebt, eW3, eb3,
           nW1, nb1, nW2, nb2, ng, nbt, nW3, nb3):
    bsz, O, OBS = states.shape
    H, ACT = eW2.shape[0], _ACT
    bb = _BB

    eW1a, eW1b = eW1[:OBS], eW1[OBS:]
    nW1x = nW1[:OBS]
    nW4 = nW1[OBS:OBS + ACT]
    nW1g = nW1[OBS + ACT:]
    act_b = jnp.broadcast_to(action[:, None], (bsz, O)).astype(jnp.int32)

    row = lambda v: v.reshape(1, -1)
    weights = (eW1a, eW1b, row(eb1), eW2, row(eb2), row(eg), row(ebt),
               eW3, row(eb3),
               nW1x, nW4, nW1g, row(nb1), nW2, row(nb2), row(ng), row(nbt),
               nW3, row(nb3))

    w_specs = [pl.BlockSpec(w.shape, lambda i: (0, 0)) for w in weights]
    in_specs = ([pl.BlockSpec((bb, O, OBS), lambda i: (i, 0, 0)),
                 pl.BlockSpec((bb, O), lambda i: (i, 0))] + w_specs)

    return pl.pallas_call(
        _fused,
        grid=(bsz // bb,),
        in_specs=in_specs,
        out_specs=pl.BlockSpec((bb, O, OBS), lambda i: (i, 0, 0)),
        out_shape=jax.ShapeDtypeStruct((bsz, O, OBS), jnp.float32),
        compiler_params=pltpu.CompilerParams(
            dimension_semantics=("parallel",)),
    )(states, act_b, *weights)


# speedup vs baseline: 21.2858x; 1.0217x over previous
import jax
import jax.numpy as jnp
from jax.experimental import pallas as pl
from jax.experimental.pallas import tpu as pltpu

_O, _OBS, _ACT, _H = 32, 32, 4, 64
_BB = 16


def _fused(x_ref, act_ref,
           eW1a_ref, eW1b_ref, eb1_ref, eW2_ref, eb2_ref, eg_ref, ebt_ref,
           eW3_ref, eb3_ref,
           nW1x_ref, nW4_ref, nW1g_ref, nb1_ref, nW2_ref, nb2_ref, ng_ref,
           nbt_ref, nW3_ref, nb3_ref, out_ref):
    bb = x_ref.shape[0]
    O, OBS, ACT, H = _O, _OBS, _ACT, _H
    f32 = jnp.float32

    x = x_ref[...].reshape(bb * O, OBS)

    a_src = jnp.dot(x, eW1a_ref[...], preferred_element_type=f32)
    b_dst = jnp.dot(x, eW1b_ref[...], preferred_element_type=f32)
    h1 = (a_src.reshape(bb, O, 1, H) + b_dst.reshape(bb, 1, O, H)
          + eb1_ref[...].reshape(1, 1, 1, H))
    h1 = jnp.maximum(h1, 0.0).reshape(bb * O * O, H)

    h2 = jnp.dot(h1, eW2_ref[...], preferred_element_type=f32) + eb2_ref[...]
    mu = jnp.mean(h2, axis=-1, keepdims=True)
    var = jnp.mean((h2 - mu) ** 2, axis=-1, keepdims=True)
    h2 = (h2 - mu) * jax.lax.rsqrt(var + 1e-5) * eg_ref[...] + ebt_ref[...]
    h2 = jnp.maximum(h2, 0.0)

    e = jnp.dot(h2, eW3_ref[...], preferred_element_type=f32) + eb3_ref[...]
    e4 = e.reshape(bb, O, O, H)

    ii = jax.lax.broadcasted_iota(jnp.int32, (1, O, O, 1), 1)
    jj = jax.lax.broadcasted_iota(jnp.int32, (1, O, O, 1), 2)
    mask = (ii != jj).astype(f32)
    agg = jnp.sum(e4 * mask, axis=2).reshape(bb * O, H)

    act = act_ref[...]
    obj_sel = (act // ACT ==
               jax.lax.broadcasted_iota(jnp.int32, (bb, O), 1)).astype(f32)
    mod = act[:, :1] % ACT
    wrow = jnp.zeros((bb, H), f32)
    for k in range(ACT):
        wrow = wrow + (mod == k).astype(f32) * nW4_ref[k:k + 1, :]
    contrib = (obj_sel.reshape(bb, O, 1) * wrow.reshape(bb, 1, H))
    contrib = contrib.reshape(bb * O, H)

    n1 = (jnp.dot(x, nW1x_ref[...], preferred_element_type=f32)
          + jnp.dot(agg, nW1g_ref[...], preferred_element_type=f32)
          + contrib + nb1_ref[...])
    n1 = jnp.maximum(n1, 0.0)
    n2 = jnp.dot(n1, nW2_ref[...], preferred_element_type=f32) + nb2_ref[...]
    mu2 = jnp.mean(n2, axis=-1, keepdims=True)
    var2 = jnp.mean((n2 - mu2) ** 2, axis=-1, keepdims=True)
    n2 = (n2 - mu2) * jax.lax.rsqrt(var2 + 1e-5) * ng_ref[...] + nbt_ref[...]
    n2 = jnp.maximum(n2, 0.0)
    out = jnp.dot(n2, nW3_ref[...], preferred_element_type=f32) + nb3_ref[...]
    out_ref[...] = out.reshape(bb, O, OBS)


def kernel(states, action, eW1, eb1, eW2, eb2, eg, ebt, eW3, eb3,
           nW1, nb1, nW2, nb2, ng, nbt, nW3, nb3):
    bsz, O, OBS = states.shape
    H, ACT = eW2.shape[0], _ACT
    bb = _BB

    eW1a, eW1b = eW1[:OBS], eW1[OBS:]
    nW1x = nW1[:OBS]
    nW4 = nW1[OBS:OBS + ACT]
    nW1g = nW1[OBS + ACT:]
    act_b = jnp.broadcast_to(action[:, None], (bsz, O)).astype(jnp.int32)

    row = lambda v: v.reshape(1, -1)
    weights = (eW1a, eW1b, row(eb1), eW2, row(eb2), row(eg), row(ebt),
               eW3, row(eb3),
               nW1x, nW4, nW1g, row(nb1), nW2, row(nb2), row(ng), row(nbt),
               nW3, row(nb3))

    w_specs = [pl.BlockSpec(w.shape, lambda i: (0, 0)) for w in weights]
    in_specs = ([pl.BlockSpec((bb, O, OBS), lambda i: (i, 0, 0)),
                 pl.BlockSpec((bb, O), lambda i: (i, 0))] + w_specs)

    return pl.pallas_call(
        _fused,
        grid=(bsz // bb,),
        in_specs=in_specs,
        out_specs=pl.BlockSpec((bb, O, OBS), lambda i: (i, 0, 0)),
        out_shape=jax.ShapeDtypeStruct((bsz, O, OBS), jnp.float32),
        compiler_params=pltpu.CompilerParams(
            dimension_semantics=("parallel",)),
    )(states, act_b, *weights)


# speedup vs baseline: 21.9369x; 1.0306x over previous
import jax
import jax.numpy as jnp
from jax.experimental import pallas as pl
from jax.experimental.pallas import tpu as pltpu

_O, _OBS, _ACT, _H = 32, 32, 4, 64
_BB = 16


def _fused(x_ref, act_ref,
           eW1a_ref, eW1b_ref, eb1_ref, eW2_ref, eb2_ref, eg_ref, ebt_ref,
           eW3_ref, eb3_ref,
           nW1x_ref, nW4_ref, nW1g_ref, nb1_ref, nW2_ref, nb2_ref, ng_ref,
           nbt_ref, nW3_ref, nb3_ref, out_ref):
    bb = x_ref.shape[0]
    O, OBS, ACT, H = _O, _OBS, _ACT, _H
    f32 = jnp.float32

    x = x_ref[...].reshape(bb * O, OBS)

    a_src = jnp.dot(x, eW1a_ref[...], preferred_element_type=f32) + eb1_ref[...]
    b_dst = jnp.dot(x, eW1b_ref[...], preferred_element_type=f32)
    h1 = a_src.reshape(bb, O, 1, H) + b_dst.reshape(bb, 1, O, H)
    h1 = jnp.maximum(h1, 0.0).reshape(bb * O * O, H)

    h2 = jnp.dot(h1, eW2_ref[...], preferred_element_type=f32) + eb2_ref[...]
    mu = jnp.mean(h2, axis=-1, keepdims=True)
    var = jnp.mean((h2 - mu) ** 2, axis=-1, keepdims=True)
    h2 = (h2 - mu) * jax.lax.rsqrt(var + 1e-5) * eg_ref[...] + ebt_ref[...]
    h2 = jnp.maximum(h2, 0.0)

    ii = jax.lax.broadcasted_iota(jnp.int32, (1, O, O, 1), 1)
    jj = jax.lax.broadcasted_iota(jnp.int32, (1, O, O, 1), 2)
    h2m = jnp.where(ii != jj, h2.reshape(bb, O, O, H), 0.0)
    hagg = jnp.sum(h2m, axis=2).reshape(bb * O, H)
    agg = (jnp.dot(hagg, eW3_ref[...], preferred_element_type=f32)
           + (O - 1) * eb3_ref[...])

    act = act_ref[...]
    obj_sel = (act // ACT ==
               jax.lax.broadcasted_iota(jnp.int32, (bb, O), 1)).astype(f32)
    mod = act[:, :1] % ACT
    wrow = jnp.zeros((bb, H), f32)
    for k in range(ACT):
        wrow = wrow + (mod == k).astype(f32) * nW4_ref[k:k + 1, :]
    contrib = (obj_sel.reshape(bb, O, 1) * wrow.reshape(bb, 1, H))
    contrib = contrib.reshape(bb * O, H)

    n1 = (jnp.dot(x, nW1x_ref[...], preferred_element_type=f32)
          + jnp.dot(agg, nW1g_ref[...], preferred_element_type=f32)
          + contrib + nb1_ref[...])
    n1 = jnp.maximum(n1, 0.0)
    n2 = jnp.dot(n1, nW2_ref[...], preferred_element_type=f32) + nb2_ref[...]
    mu2 = jnp.mean(n2, axis=-1, keepdims=True)
    var2 = jnp.mean((n2 - mu2) ** 2, axis=-1, keepdims=True)
    n2 = (n2 - mu2) * jax.lax.rsqrt(var2 + 1e-5) * ng_ref[...] + nbt_ref[...]
    n2 = jnp.maximum(n2, 0.0)
    out = jnp.dot(n2, nW3_ref[...], preferred_element_type=f32) + nb3_ref[...]
    out_ref[...] = out.reshape(bb, O, OBS)


def kernel(states, action, eW1, eb1, eW2, eb2, eg, ebt, eW3, eb3,
           nW1, nb1, nW2, nb2, ng, nbt, nW3, nb3):
    bsz, O, OBS = states.shape
    H, ACT = eW2.shape[0], _ACT
    bb = _BB

    eW1a, eW1b = eW1[:OBS], eW1[OBS:]
    nW1x = nW1[:OBS]
    nW4 = nW1[OBS:OBS + ACT]
    nW1g = nW1[OBS + ACT:]
    act_b = jnp.broadcast_to(action[:, None], (bsz, O)).astype(jnp.int32)

    row = lambda v: v.reshape(1, -1)
    weights = (eW1a, eW1b, row(eb1), eW2, row(eb2), row(eg), row(ebt),
               eW3, row(eb3),
               nW1x, nW4, nW1g, row(nb1), nW2, row(nb2), row(ng), row(nbt),
               nW3, row(nb3))

    w_specs = [pl.BlockSpec(w.shape, lambda i: (0, 0)) for w in weights]
    in_specs = ([pl.BlockSpec((bb, O, OBS), lambda i: (i, 0, 0)),
                 pl.BlockSpec((bb, O), lambda i: (i, 0))] + w_specs)

    return pl.pallas_call(
        _fused,
        grid=(bsz // bb,),
        in_specs=in_specs,
        out_specs=pl.BlockSpec((bb, O, OBS), lambda i: (i, 0, 0)),
        out_shape=jax.ShapeDtypeStruct((bsz, O, OBS), jnp.float32),
        compiler_params=pltpu.CompilerParams(
            dimension_semantics=("parallel",)),
    )(states, act_b, *weights)
